# baseline (speedup 1.0000x reference)
import functools

import jax
import jax.numpy as jnp
from jax import lax
from jax.experimental import pallas as pl
from jax.experimental.pallas import tpu as pltpu
from jax.experimental.pallas import tpu_sc as plsc

_N_ITERS = 5
_H = 224
_W = 224
_C = 192
_HW = _H * _W
_NSUB = 16
_ROWS_T = _H // _NSUB
_PXT = _ROWS_T * _W
_CP = 256
_P = 64
_NCH = _PXT // _P
_GPC = _P // 16
_NG = _PXT // 16
_NU = _C // 16


def _precompute_fields(B):
    radius0 = max(_H, _W) // 2
    pack0, dyx = [], []
    for i in range(B):
        pm_key = jax.random.fold_in(jax.random.key(42), i)
        ky, kx = jax.random.split(pm_key)
        sy = jax.random.randint(ky, (_H, _W), 0, _H)
        sx = jax.random.randint(kx, (_H, _W), 0, _W)
        pack0.append((sy * 256 + sx).astype(jnp.int32).ravel())
        per_it = []
        for it in range(_N_ITERS):
            r = max(radius0 >> it, 1)
            ss = []
            for s2 in range(2):
                kk = jax.random.fold_in(pm_key, it * 97 + s2 + 1)
                k1, k2 = jax.random.split(kk)
                dy = jax.random.randint(k1, (_H, _W), -r, r + 1)
                dx = jax.random.randint(k2, (_H, _W), -r, r + 1)
                ss.append(jnp.stack([dy.ravel(), dx.ravel()]))
            per_it.append(jnp.stack(ss))
        dyx.append(jnp.stack(per_it))
    pack0 = jnp.concatenate(pack0)
    dyx = jnp.stack(dyx)
    dyx = dyx.transpose(1, 2, 3, 0, 4).reshape(_N_ITERS * 2 * 2 * B * _HW)
    return pack0, dyx.astype(jnp.int32)


def _row_cost_acc(qb, kg, p):
    d0 = qb[p, pl.ds(0, 16)] - kg[p, pl.ds(0, 16)]
    acc = d0 * d0
    for u in range(1, _NU):
        d = qb[p, pl.ds(u * 16, 16)] - kg[p, pl.ds(u * 16, 16)]
        acc = acc + d * d
    return acc


def _pm_body(q_hbm, k_hbm, v_hbm, p0_hbm, dyx_hbm, out_hbm,
             st, cpack, fidx, cur, best, recp, recc, qb, kg, dybuf, accT,
             state_sm, sem):
    c = lax.axis_index("c")
    s = lax.axis_index("s")
    gbase = c * _HW + s * _PXT
    coff = c * _HW
    iota = lax.iota(jnp.int32, 16)

    def unpack(p):
        return jnp.right_shift(p, 8), jnp.bitwise_and(p, 255)

    def fidx_from(ref, off_of_g):
        def fb(g, _):
            pk = ref[pl.ds(off_of_g(g), 16)]
            sy, sx = unpack(pk)
            fidx[pl.ds(g * 16, 16)] = sy * _W + sx + coff
            return 0
        lax.fori_loop(0, _GPC, fb, 0)

    def eval_chunks(j, update):
        def cbody(ch, _):
            pbase = ch * _P
            pltpu.sync_copy(q_hbm.at[pl.ds(gbase + pbase, _P)], qb)
            if update:
                fidx_from(cpack, lambda g: j * _PXT + pbase + g * 16)
            else:
                fidx_from(cur, lambda g: pbase + g * 16)
            pltpu.async_copy(k_hbm.at[fidx], kg, sem).wait()

            def pbody(p, _2):
                acc = _row_cost_acc(qb, kg, p)
                accT[pl.ds(p * 16, 16)] = acc
                return 0
            lax.fori_loop(0, _P, pbody, 0)

            def gred(g, _2):
                base16 = (g * 16 + iota) * 16
                tot = plsc.load_gather(accT, [base16])
                for lane in range(1, 16):
                    tot = tot + plsc.load_gather(accT, [base16 + lane])
                lsl = pl.ds(pbase + g * 16, 16)
                if update:
                    b0 = best[lsl]
                    bet = tot < b0
                    best[lsl] = jnp.where(bet, tot, b0)
                    cur[lsl] = jnp.where(
                        bet, cpack[pl.ds(j * _PXT + pbase + g * 16, 16)],
                        cur[lsl])
                else:
                    best[lsl] = tot
                return 0
            lax.fori_loop(0, _GPC, gred, 0)
            return 0
        lax.fori_loop(0, _NCH, cbody, 0)

    pltpu.sync_copy(p0_hbm.at[pl.ds(gbase, _PXT)], cur)
    eval_chunks(-1, False)
    pltpu.sync_copy(cur, state_sm.at[pl.ds(s * _PXT, _PXT)])
    plsc.subcore_barrier()

    def iteration(it, _):
        top = jnp.where(s == 0, _H - 1, s * _ROWS_T - 1)
        bot = jnp.where(s == _NSUB - 1, 0, (s + 1) * _ROWS_T)
        pltpu.sync_copy(state_sm.at[pl.ds(top * _W, _W)], st.at[pl.ds(0, _W)])
        pltpu.sync_copy(state_sm.at[pl.ds(s * _PXT, _PXT)],
                        st.at[pl.ds(_W, _PXT)])
        pltpu.sync_copy(state_sm.at[pl.ds(bot * _W, _W)],
                        st.at[pl.ds(_W + _PXT, _W)])
        plsc.subcore_barrier()

        for s2 in range(2):
            for comp in range(2):
                src = (it * 4 + s2 * 2 + comp) * (2 * _HW) + gbase
                pltpu.sync_copy(
                    dyx_hbm.at[pl.ds(src, _PXT)],
                    dybuf.at[pl.ds((s2 * 2 + comp) * _PXT, _PXT)])

        def cand_body(g, _2):
            lr = g // _ROWS_T
            gx = g - lr * _ROWS_T
            x0 = gx * 16
            introw = (lr + 1) * _W
            base = g * 16
            offm = iota + (x0 - 1)
            offm = jnp.where(offm < 0, offm + _W, offm)
            pL = plsc.load_gather(st, [introw + offm])
            syL, sxL = unpack(pL)
            cpack[pl.ds(0 * _PXT + base, 16)] = (
                jnp.left_shift(syL, 8) + jnp.minimum(sxL + 1, _W - 1))
            offp = iota + (x0 + 1)
            offp = jnp.where(offp > _W - 1, offp - _W, offp)
            pR = plsc.load_gather(st, [introw + offp])
            syR, sxR = unpack(pR)
            cpack[pl.ds(1 * _PXT + base, 16)] = (
                jnp.left_shift(syR, 8) + jnp.maximum(sxR - 1, 0))
            pU = st[pl.ds(lr * _W + x0, 16)]
            syU, sxU = unpack(pU)
            cpack[pl.ds(2 * _PXT + base, 16)] = (
                jnp.left_shift(jnp.minimum(syU + 1, _H - 1), 8) + sxU)
            pD = st[pl.ds((lr + 2) * _W + x0, 16)]
            syD, sxD = unpack(pD)
            cpack[pl.ds(3 * _PXT + base, 16)] = (
                jnp.left_shift(jnp.maximum(syD - 1, 0), 8) + sxD)
            pC = st[pl.ds(introw + x0, 16)]
            syC, sxC = unpack(pC)
            for s2 in range(2):
                dy = dybuf[pl.ds((s2 * 2 + 0) * _PXT + base, 16)]
                dx = dybuf[pl.ds((s2 * 2 + 1) * _PXT + base, 16)]
                cy = jnp.minimum(jnp.maximum(syC + dy, 0), _H - 1)
                cx = jnp.minimum(jnp.maximum(sxC + dx, 0), _W - 1)
                cpack[pl.ds((4 + s2) * _PXT + base, 16)] = (
                    jnp.left_shift(cy, 8) + cx)
            return 0
        lax.fori_loop(0, _NG, cand_body, 0)

        for j in range(6):
            eval_chunks(j, True)

        def rec_body(g, _2):
            recp[pl.ds(it * _PXT + g * 16, 16)] = cur[pl.ds(g * 16, 16)]
            recc[pl.ds(it * _PXT + g * 16, 16)] = best[pl.ds(g * 16, 16)]
            return 0
        lax.fori_loop(0, _NG, rec_body, 0)

        pltpu.sync_copy(cur, state_sm.at[pl.ds(s * _PXT, _PXT)])
        plsc.subcore_barrier()
        return 0
    lax.fori_loop(0, _N_ITERS, iteration, 0)

    def soft_body(g, _):
        base = g * 16
        cs = [recc[pl.ds(j * _PXT + base, 16)] for j in range(_N_ITERS)]
        m = cs[0]
        for j in range(1, _N_ITERS):
            m = jnp.minimum(m, cs[j])
        es = [jnp.exp(m - cj) for cj in cs]
        tot = es[0]
        for j in range(1, _N_ITERS):
            tot = tot + es[j]
        inv = 1.0 / tot
        for j in range(_N_ITERS):
            recc[pl.ds(j * _PXT + base, 16)] = es[j] * inv
        return 0
    lax.fori_loop(0, _NG, soft_body, 0)

    def obody(ch, _):
        pbase = ch * _P
        for j in range(_N_ITERS):
            fidx_from(recp, lambda g: j * _PXT + pbase + g * 16)
            pltpu.async_copy(v_hbm.at[fidx], kg, sem).wait()

            def px(p, _2):
                widx = jnp.broadcast_to(j * _PXT + pbase + p, (16,))
                w = plsc.load_gather(recc, [widx])
                for u in range(_NU):
                    sl = pl.ds(u * 16, 16)
                    if j == 0:
                        qb[p, sl] = w * kg[p, sl]
                    else:
                        qb[p, sl] = qb[p, sl] + w * kg[p, sl]
                return 0
            lax.fori_loop(0, _P, px, 0)
        pltpu.sync_copy(qb, out_hbm.at[pl.ds(gbase + pbase, _P)])
        return 0
    lax.fori_loop(0, _NCH, obody, 0)


def kernel(q, k, v):
    B, C, H, W = q.shape
    pack0, dyx = _precompute_fields(B)
    q2 = q.transpose(0, 2, 3, 1).reshape(B * _HW, C)
    k2 = jnp.pad(k.transpose(0, 2, 3, 1).reshape(B * _HW, C),
                 ((0, 0), (0, _CP - C)))
    v2 = jnp.pad(v.transpose(0, 2, 3, 1).reshape(B * _HW, C),
                 ((0, 0), (0, _CP - C)))

    mesh = plsc.VectorSubcoreMesh(core_axis_name="c", subcore_axis_name="s")
    f32, i32 = jnp.float32, jnp.int32
    pm = pl.kernel(
        _pm_body,
        mesh=mesh,
        out_type=jax.ShapeDtypeStruct((B * _HW, C), f32),
        scratch_types=[
            pltpu.VMEM((_W * (_ROWS_T + 2),), i32),
            pltpu.VMEM((6 * _PXT,), i32),
            pltpu.VMEM((_P,), i32),
            pltpu.VMEM((_PXT,), i32),
            pltpu.VMEM((_PXT,), f32),
            pltpu.VMEM((_N_ITERS * _PXT,), i32),
            pltpu.VMEM((_N_ITERS * _PXT,), f32),
            pltpu.VMEM((_P, _C), f32),
            pltpu.VMEM((_P, _CP), f32),
            pltpu.VMEM((4 * _PXT,), i32),
            pltpu.VMEM((16 * _P,), f32),
            pltpu.VMEM_SHARED((_HW,), i32),
            pltpu.SemaphoreType.DMA,
        ],
        compiler_params=pltpu.CompilerParams(needs_layout_passes=False),
    )
    out2 = pm(q2, k2, v2, pack0, dyx)
    return out2.reshape(B, H, W, C).transpose(0, 3, 1, 2)


# speedup vs baseline: 1.5457x; 1.5457x over previous
import functools

import jax
import jax.numpy as jnp
from jax import lax
from jax.experimental import pallas as pl
from jax.experimental.pallas import tpu as pltpu
from jax.experimental.pallas import tpu_sc as plsc

_N_ITERS = 5
_H = 224
_W = 224
_C = 192
_HW = _H * _W
_NSUB = 16
_ROWS_T = _H // _NSUB
_PXT = _ROWS_T * _W
_CP = 256
_P = 64
_NCH = _PXT // _P
_GPC = _P // 16
_NG = _PXT // 16
_NU = _C // 16


def _precompute_fields(B):
    radius0 = max(_H, _W) // 2
    pack0, dyx = [], []
    for i in range(B):
        pm_key = jax.random.fold_in(jax.random.key(42), i)
        ky, kx = jax.random.split(pm_key)
        sy = jax.random.randint(ky, (_H, _W), 0, _H)
        sx = jax.random.randint(kx, (_H, _W), 0, _W)
        pack0.append((sy * 256 + sx).astype(jnp.int32).ravel())
        per_it = []
        for it in range(_N_ITERS):
            r = max(radius0 >> it, 1)
            ss = []
            for s2 in range(2):
                kk = jax.random.fold_in(pm_key, it * 97 + s2 + 1)
                k1, k2 = jax.random.split(kk)
                dy = jax.random.randint(k1, (_H, _W), -r, r + 1)
                dx = jax.random.randint(k2, (_H, _W), -r, r + 1)
                ss.append(jnp.stack([dy.ravel(), dx.ravel()]))
            per_it.append(jnp.stack(ss))
        dyx.append(jnp.stack(per_it))
    pack0 = jnp.concatenate(pack0)
    dyx = jnp.stack(dyx)
    dyx = dyx.transpose(1, 2, 3, 0, 4).reshape(_N_ITERS * 2 * 2 * B * _HW)
    return pack0, dyx.astype(jnp.int32)


def _row_cost_acc(qb, kg, p):
    d0 = qb[p, pl.ds(0, 16)] - kg[p, pl.ds(0, 16)]
    acc = d0 * d0
    for u in range(1, _NU):
        d = qb[p, pl.ds(u * 16, 16)] - kg[p, pl.ds(u * 16, 16)]
        acc = acc + d * d
    return acc


def _pm_body(q_hbm, k_hbm, v_hbm, p0_hbm, dyx_hbm, out_hbm,
             st, cpack, fidx0, fidx1, cur, best, recp, recc, qb, kg0, kg1,
             dybuf, accT, state_sm, sem0, sem1):
    c = lax.axis_index("c")
    s = lax.axis_index("s")
    gbase = c * _HW + s * _PXT
    coff = c * _HW
    iota = lax.iota(jnp.int32, 16)

    fidxs = (fidx0, fidx1)
    kgs = (kg0, kg1)
    sems = (sem0, sem1)

    def unpack(p):
        return jnp.right_shift(p, 8), jnp.bitwise_and(p, 255)

    def start_gather(tbl, src, src_off, slot):
        fb = fidxs[slot]
        for g in range(_GPC):
            pk = src[pl.ds(src_off + g * 16, 16)]
            sy, sx = unpack(pk)
            fb[pl.ds(g * 16, 16)] = sy * _W + sx + coff
        return pltpu.async_copy(tbl.at[fb], kgs[slot], sems[slot])

    def partial_costs(kg):
        def pbody(p, _2):
            acc = _row_cost_acc(qb, kg, p)
            accT[pl.ds(p * 16, 16)] = acc
            return 0
        lax.fori_loop(0, _P, pbody, 0, unroll=2)

    def reduce_and_select(pbase, j, update):
        for g in range(_GPC):
            base16 = (g * 16 + iota) * 16
            tot = plsc.load_gather(accT, [base16])
            for lane in range(1, 16):
                tot = tot + plsc.load_gather(accT, [base16 + lane])
            lsl = pl.ds(pbase + g * 16, 16)
            if update:
                b0 = best[lsl]
                bet = tot < b0
                best[lsl] = jnp.where(bet, tot, b0)
                cur[lsl] = jnp.where(
                    bet, cpack[pl.ds(j * _PXT + pbase + g * 16, 16)],
                    cur[lsl])
            else:
                best[lsl] = tot

    def eval_chunks_iter():
        def cbody(ch, _):
            pbase = ch * _P
            pltpu.sync_copy(q_hbm.at[pl.ds(gbase + pbase, _P)], qb)
            pend = start_gather(k_hbm, cpack, 0 * _PXT + pbase, 0)
            for j in range(6):
                slot = j & 1
                nxt = (start_gather(k_hbm, cpack, (j + 1) * _PXT + pbase,
                                    1 - slot) if j < 5 else None)
                pend.wait()
                partial_costs(kgs[slot])
                reduce_and_select(pbase, j, True)
                pend = nxt
            return 0
        lax.fori_loop(0, _NCH, cbody, 0)

    pltpu.sync_copy(p0_hbm.at[pl.ds(gbase, _PXT)], cur)

    def init_body(ch, _):
        pbase = ch * _P
        pltpu.sync_copy(q_hbm.at[pl.ds(gbase + pbase, _P)], qb)
        start_gather(k_hbm, cur, pbase, 0).wait()
        partial_costs(kg0)
        reduce_and_select(pbase, 0, False)
        return 0
    lax.fori_loop(0, _NCH, init_body, 0)
    pltpu.sync_copy(cur, state_sm.at[pl.ds(s * _PXT, _PXT)])
    plsc.subcore_barrier()

    def iteration(it, _):
        top = jnp.where(s == 0, _H - 1, s * _ROWS_T - 1)
        bot = jnp.where(s == _NSUB - 1, 0, (s + 1) * _ROWS_T)
        pltpu.sync_copy(state_sm.at[pl.ds(top * _W, _W)], st.at[pl.ds(0, _W)])
        pltpu.sync_copy(state_sm.at[pl.ds(s * _PXT, _PXT)],
                        st.at[pl.ds(_W, _PXT)])
        pltpu.sync_copy(state_sm.at[pl.ds(bot * _W, _W)],
                        st.at[pl.ds(_W + _PXT, _W)])
        plsc.subcore_barrier()

        for s2 in range(2):
            for comp in range(2):
                src = (it * 4 + s2 * 2 + comp) * (2 * _HW) + gbase
                pltpu.sync_copy(
                    dyx_hbm.at[pl.ds(src, _PXT)],
                    dybuf.at[pl.ds((s2 * 2 + comp) * _PXT, _PXT)])

        def cand_body(g, _2):
            lr = g // _ROWS_T
            gx = g - lr * _ROWS_T
            x0 = gx * 16
            introw = (lr + 1) * _W
            base = g * 16
            offm = iota + (x0 - 1)
            offm = jnp.where(offm < 0, offm + _W, offm)
            pL = plsc.load_gather(st, [introw + offm])
            syL, sxL = unpack(pL)
            cpack[pl.ds(0 * _PXT + base, 16)] = (
                jnp.left_shift(syL, 8) + jnp.minimum(sxL + 1, _W - 1))
            offp = iota + (x0 + 1)
            offp = jnp.where(offp > _W - 1, offp - _W, offp)
            pR = plsc.load_gather(st, [introw + offp])
            syR, sxR = unpack(pR)
            cpack[pl.ds(1 * _PXT + base, 16)] = (
                jnp.left_shift(syR, 8) + jnp.maximum(sxR - 1, 0))
            pU = st[pl.ds(lr * _W + x0, 16)]
            syU, sxU = unpack(pU)
            cpack[pl.ds(2 * _PXT + base, 16)] = (
                jnp.left_shift(jnp.minimum(syU + 1, _H - 1), 8) + sxU)
            pD = st[pl.ds((lr + 2) * _W + x0, 16)]
            syD, sxD = unpack(pD)
            cpack[pl.ds(3 * _PXT + base, 16)] = (
                jnp.left_shift(jnp.maximum(syD - 1, 0), 8) + sxD)
            pC = st[pl.ds(introw + x0, 16)]
            syC, sxC = unpack(pC)
            for s2 in range(2):
                dy = dybuf[pl.ds((s2 * 2 + 0) * _PXT + base, 16)]
                dx = dybuf[pl.ds((s2 * 2 + 1) * _PXT + base, 16)]
                cy = jnp.minimum(jnp.maximum(syC + dy, 0), _H - 1)
                cx = jnp.minimum(jnp.maximum(sxC + dx, 0), _W - 1)
                cpack[pl.ds((4 + s2) * _PXT + base, 16)] = (
                    jnp.left_shift(cy, 8) + cx)
            return 0
        lax.fori_loop(0, _NG, cand_body, 0)

        eval_chunks_iter()

        def rec_body(g, _2):
            recp[pl.ds(it * _PXT + g * 16, 16)] = cur[pl.ds(g * 16, 16)]
            recc[pl.ds(it * _PXT + g * 16, 16)] = best[pl.ds(g * 16, 16)]
            return 0
        lax.fori_loop(0, _NG, rec_body, 0)

        pltpu.sync_copy(cur, state_sm.at[pl.ds(s * _PXT, _PXT)])
        plsc.subcore_barrier()
        return 0
    lax.fori_loop(0, _N_ITERS, iteration, 0)

    def soft_body(g, _):
        base = g * 16
        cs = [recc[pl.ds(j * _PXT + base, 16)] for j in range(_N_ITERS)]
        m = cs[0]
        for j in range(1, _N_ITERS):
            m = jnp.minimum(m, cs[j])
        es = [jnp.exp(m - cj) for cj in cs]
        tot = es[0]
        for j in range(1, _N_ITERS):
            tot = tot + es[j]
        inv = 1.0 / tot
        for j in range(_N_ITERS):
            recc[pl.ds(j * _PXT + base, 16)] = es[j] * inv
        return 0
    lax.fori_loop(0, _NG, soft_body, 0)

    def obody(ch, _):
        pbase = ch * _P
        pend = start_gather(v_hbm, recp, 0 * _PXT + pbase, 0)
        for j in range(_N_ITERS):
            slot = j & 1
            nxt = (start_gather(v_hbm, recp, (j + 1) * _PXT + pbase,
                                1 - slot) if j < _N_ITERS - 1 else None)
            pend.wait()
            kg = kgs[slot]

            def px(p, _2, kg=kg, j=j):
                widx = jnp.broadcast_to(j * _PXT + pbase + p, (16,))
                w = plsc.load_gather(recc, [widx])
                for u in range(_NU):
                    sl = pl.ds(u * 16, 16)
                    if j == 0:
                        qb[p, sl] = w * kg[p, sl]
                    else:
                        qb[p, sl] = qb[p, sl] + w * kg[p, sl]
                return 0
            lax.fori_loop(0, _P, px, 0, unroll=2)
            pend = nxt
        pltpu.sync_copy(qb, out_hbm.at[pl.ds(gbase + pbase, _P)])
        return 0
    lax.fori_loop(0, _NCH, obody, 0)


def kernel(q, k, v):
    B, C, H, W = q.shape
    pack0, dyx = _precompute_fields(B)
    q2 = q.transpose(0, 2, 3, 1).reshape(B * _HW, C)
    k2 = jnp.pad(k.transpose(0, 2, 3, 1).reshape(B * _HW, C),
                 ((0, 0), (0, _CP - C)))
    v2 = jnp.pad(v.transpose(0, 2, 3, 1).reshape(B * _HW, C),
                 ((0, 0), (0, _CP - C)))

    mesh = plsc.VectorSubcoreMesh(core_axis_name="c", subcore_axis_name="s")
    f32, i32 = jnp.float32, jnp.int32
    pm = pl.kernel(
        _pm_body,
        mesh=mesh,
        out_type=jax.ShapeDtypeStruct((B * _HW, C), f32),
        scratch_types=[
            pltpu.VMEM((_W * (_ROWS_T + 2),), i32),
            pltpu.VMEM((6 * _PXT,), i32),
            pltpu.VMEM((_P,), i32),
            pltpu.VMEM((_P,), i32),
            pltpu.VMEM((_PXT,), i32),
            pltpu.VMEM((_PXT,), f32),
            pltpu.VMEM((_N_ITERS * _PXT,), i32),
            pltpu.VMEM((_N_ITERS * _PXT,), f32),
            pltpu.VMEM((_P, _C), f32),
            pltpu.VMEM((_P, _CP), f32),
            pltpu.VMEM((_P, _CP), f32),
            pltpu.VMEM((4 * _PXT,), i32),
            pltpu.VMEM((16 * _P,), f32),
            pltpu.VMEM_SHARED((_HW,), i32),
            pltpu.SemaphoreType.DMA,
            pltpu.SemaphoreType.DMA,
        ],
        compiler_params=pltpu.CompilerParams(needs_layout_passes=False),
    )
    out2 = pm(q2, k2, v2, pack0, dyx)
    return out2.reshape(B, H, W, C).transpose(0, 3, 1, 2)


# speedup vs baseline: 1.5697x; 1.0156x over previous
import functools

import jax
import jax.numpy as jnp
from jax import lax
from jax.experimental import pallas as pl
from jax.experimental.pallas import tpu as pltpu
from jax.experimental.pallas import tpu_sc as plsc

_N_ITERS = 5
_H = 224
_W = 224
_C = 192
_HW = _H * _W
_NSUB = 16
_ROWS_T = _H // _NSUB
_PXT = _ROWS_T * _W
_CP = 256
_P = 64
_NCH = _PXT // _P
_GPC = _P // 16
_NG = _PXT // 16
_NU = _C // 16


def _precompute_fields(B):
    radius0 = max(_H, _W) // 2
    pack0, dyx = [], []
    for i in range(B):
        pm_key = jax.random.fold_in(jax.random.key(42), i)
        ky, kx = jax.random.split(pm_key)
        sy = jax.random.randint(ky, (_H, _W), 0, _H)
        sx = jax.random.randint(kx, (_H, _W), 0, _W)
        pack0.append((sy * 256 + sx).astype(jnp.int32).ravel())
        per_it = []
        for it in range(_N_ITERS):
            r = max(radius0 >> it, 1)
            ss = []
            for s2 in range(2):
                kk = jax.random.fold_in(pm_key, it * 97 + s2 + 1)
                k1, k2 = jax.random.split(kk)
                dy = jax.random.randint(k1, (_H, _W), -r, r + 1)
                dx = jax.random.randint(k2, (_H, _W), -r, r + 1)
                ss.append(jnp.stack([dy.ravel(), dx.ravel()]))
            per_it.append(jnp.stack(ss))
        dyx.append(jnp.stack(per_it))
    pack0 = jnp.concatenate(pack0)
    dyx = jnp.stack(dyx)
    dyx = dyx.transpose(1, 2, 3, 0, 4).reshape(_N_ITERS * 2 * 2 * B * _HW)
    return pack0, dyx.astype(jnp.int32)


def _row_cost_acc(qb, kg, p):
    d0 = qb[p, pl.ds(0, 16)] - kg[p, pl.ds(0, 16)]
    acc = d0 * d0
    for u in range(1, _NU):
        d = qb[p, pl.ds(u * 16, 16)] - kg[p, pl.ds(u * 16, 16)]
        acc = acc + d * d
    return acc


def _pm_body(q_hbm, k_hbm, v_hbm, p0_hbm, dyx_hbm, out_hbm,
             st, cpack, fidx0, fidx1, cur, best, recp, recc, qb, kg0, kg1,
             dybuf, accT, state_sm, sem0, sem1):
    c = lax.axis_index("c")
    s = lax.axis_index("s")
    gbase = c * _HW + s * _PXT
    coff = c * _HW
    iota = lax.iota(jnp.int32, 16)

    fidxs = (fidx0, fidx1)
    kgs = (kg0, kg1)
    sems = (sem0, sem1)

    def unpack(p):
        return jnp.right_shift(p, 8), jnp.bitwise_and(p, 255)

    def start_gather(tbl, src, src_off, slot):
        fb = fidxs[slot]
        for g in range(_GPC):
            pk = src[pl.ds(src_off + g * 16, 16)]
            sy, sx = unpack(pk)
            fb[pl.ds(g * 16, 16)] = sy * _W + sx + coff
        return pltpu.async_copy(tbl.at[fb], kgs[slot], sems[slot])

    def partial_costs(kg):
        def pbody(p, _2):
            acc = _row_cost_acc(qb, kg, p)
            accT[pl.ds(p * 16, 16)] = acc
            return 0
        lax.fori_loop(0, _P, pbody, 0, unroll=4)

    def reduce_and_select(pbase, j, update):
        for g in range(_GPC):
            base16 = (g * 16 + iota) * 16
            tot = plsc.load_gather(accT, [base16])
            for lane in range(1, 16):
                tot = tot + plsc.load_gather(accT, [base16 + lane])
            lsl = pl.ds(pbase + g * 16, 16)
            if update:
                b0 = best[lsl]
                bet = tot < b0
                best[lsl] = jnp.where(bet, tot, b0)
                cur[lsl] = jnp.where(
                    bet, cpack[pl.ds(j * _PXT + pbase + g * 16, 16)],
                    cur[lsl])
            else:
                best[lsl] = tot

    def eval_chunks_iter():
        def cbody(ch, _):
            pbase = ch * _P
            pend = start_gather(k_hbm, cpack, 0 * _PXT + pbase, 0)
            pltpu.sync_copy(q_hbm.at[pl.ds(gbase + pbase, _P)], qb)
            for j in range(6):
                slot = j & 1
                nxt = (start_gather(k_hbm, cpack, (j + 1) * _PXT + pbase,
                                    1 - slot) if j < 5 else None)
                pend.wait()
                partial_costs(kgs[slot])
                reduce_and_select(pbase, j, True)
                pend = nxt
            return 0
        lax.fori_loop(0, _NCH, cbody, 0)

    pltpu.sync_copy(p0_hbm.at[pl.ds(gbase, _PXT)], cur)

    def init_body(ch, _):
        pbase = ch * _P
        cp = start_gather(k_hbm, cur, pbase, 0)
        pltpu.sync_copy(q_hbm.at[pl.ds(gbase + pbase, _P)], qb)
        cp.wait()
        partial_costs(kg0)
        reduce_and_select(pbase, 0, False)
        return 0
    lax.fori_loop(0, _NCH, init_body, 0)
    pltpu.sync_copy(cur, state_sm.at[pl.ds(s * _PXT, _PXT)])
    plsc.subcore_barrier()

    def iteration(it, _):
        top = jnp.where(s == 0, _H - 1, s * _ROWS_T - 1)
        bot = jnp.where(s == _NSUB - 1, 0, (s + 1) * _ROWS_T)
        pltpu.sync_copy(state_sm.at[pl.ds(top * _W, _W)], st.at[pl.ds(0, _W)])
        pltpu.sync_copy(state_sm.at[pl.ds(s * _PXT, _PXT)],
                        st.at[pl.ds(_W, _PXT)])
        pltpu.sync_copy(state_sm.at[pl.ds(bot * _W, _W)],
                        st.at[pl.ds(_W + _PXT, _W)])
        plsc.subcore_barrier()

        for s2 in range(2):
            for comp in range(2):
                src = (it * 4 + s2 * 2 + comp) * (2 * _HW) + gbase
                pltpu.sync_copy(
                    dyx_hbm.at[pl.ds(src, _PXT)],
                    dybuf.at[pl.ds((s2 * 2 + comp) * _PXT, _PXT)])

        def cand_body(g, _2):
            lr = g // _ROWS_T
            gx = g - lr * _ROWS_T
            x0 = gx * 16
            introw = (lr + 1) * _W
            base = g * 16
            offm = iota + (x0 - 1)
            offm = jnp.where(offm < 0, offm + _W, offm)
            pL = plsc.load_gather(st, [introw + offm])
            syL, sxL = unpack(pL)
            cpack[pl.ds(0 * _PXT + base, 16)] = (
                jnp.left_shift(syL, 8) + jnp.minimum(sxL + 1, _W - 1))
            offp = iota + (x0 + 1)
            offp = jnp.where(offp > _W - 1, offp - _W, offp)
            pR = plsc.load_gather(st, [introw + offp])
            syR, sxR = unpack(pR)
            cpack[pl.ds(1 * _PXT + base, 16)] = (
                jnp.left_shift(syR, 8) + jnp.maximum(sxR - 1, 0))
            pU = st[pl.ds(lr * _W + x0, 16)]
            syU, sxU = unpack(pU)
            cpack[pl.ds(2 * _PXT + base, 16)] = (
                jnp.left_shift(jnp.minimum(syU + 1, _H - 1), 8) + sxU)
            pD = st[pl.ds((lr + 2) * _W + x0, 16)]
            syD, sxD = unpack(pD)
            cpack[pl.ds(3 * _PXT + base, 16)] = (
                jnp.left_shift(jnp.maximum(syD - 1, 0), 8) + sxD)
            pC = st[pl.ds(introw + x0, 16)]
            syC, sxC = unpack(pC)
            for s2 in range(2):
                dy = dybuf[pl.ds((s2 * 2 + 0) * _PXT + base, 16)]
                dx = dybuf[pl.ds((s2 * 2 + 1) * _PXT + base, 16)]
                cy = jnp.minimum(jnp.maximum(syC + dy, 0), _H - 1)
                cx = jnp.minimum(jnp.maximum(sxC + dx, 0), _W - 1)
                cpack[pl.ds((4 + s2) * _PXT + base, 16)] = (
                    jnp.left_shift(cy, 8) + cx)
            return 0
        lax.fori_loop(0, _NG, cand_body, 0)

        eval_chunks_iter()

        def rec_body(g, _2):
            recp[pl.ds(it * _PXT + g * 16, 16)] = cur[pl.ds(g * 16, 16)]
            recc[pl.ds(it * _PXT + g * 16, 16)] = best[pl.ds(g * 16, 16)]
            return 0
        lax.fori_loop(0, _NG, rec_body, 0)

        pltpu.sync_copy(cur, state_sm.at[pl.ds(s * _PXT, _PXT)])
        plsc.subcore_barrier()
        return 0
    lax.fori_loop(0, _N_ITERS, iteration, 0)

    def soft_body(g, _):
        base = g * 16
        cs = [recc[pl.ds(j * _PXT + base, 16)] for j in range(_N_ITERS)]
        m = cs[0]
        for j in range(1, _N_ITERS):
            m = jnp.minimum(m, cs[j])
        es = [jnp.exp(m - cj) for cj in cs]
        tot = es[0]
        for j in range(1, _N_ITERS):
            tot = tot + es[j]
        inv = 1.0 / tot
        for j in range(_N_ITERS):
            recc[pl.ds(j * _PXT + base, 16)] = es[j] * inv
        return 0
    lax.fori_loop(0, _NG, soft_body, 0)

    def obody(ch, _):
        pbase = ch * _P
        pend = start_gather(v_hbm, recp, 0 * _PXT + pbase, 0)
        for j in range(_N_ITERS):
            slot = j & 1
            nxt = (start_gather(v_hbm, recp, (j + 1) * _PXT + pbase,
                                1 - slot) if j < _N_ITERS - 1 else None)
            pend.wait()
            kg = kgs[slot]

            def px(p, _2, kg=kg, j=j):
                widx = jnp.broadcast_to(j * _PXT + pbase + p, (16,))
                w = plsc.load_gather(recc, [widx])
                for u in range(_NU):
                    sl = pl.ds(u * 16, 16)
                    if j == 0:
                        qb[p, sl] = w * kg[p, sl]
                    else:
                        qb[p, sl] = qb[p, sl] + w * kg[p, sl]
                return 0
            lax.fori_loop(0, _P, px, 0, unroll=4)
            pend = nxt
        pltpu.sync_copy(qb, out_hbm.at[pl.ds(gbase + pbase, _P)])
        return 0
    lax.fori_loop(0, _NCH, obody, 0)


def kernel(q, k, v):
    B, C, H, W = q.shape
    pack0, dyx = _precompute_fields(B)
    q2 = q.transpose(0, 2, 3, 1).reshape(B * _HW, C)
    k2 = jnp.pad(k.transpose(0, 2, 3, 1).reshape(B * _HW, C),
                 ((0, 0), (0, _CP - C)))
    v2 = jnp.pad(v.transpose(0, 2, 3, 1).reshape(B * _HW, C),
                 ((0, 0), (0, _CP - C)))

    mesh = plsc.VectorSubcoreMesh(core_axis_name="c", subcore_axis_name="s")
    f32, i32 = jnp.float32, jnp.int32
    pm = pl.kernel(
        _pm_body,
        mesh=mesh,
        out_type=jax.ShapeDtypeStruct((B * _HW, C), f32),
        scratch_types=[
            pltpu.VMEM((_W * (_ROWS_T + 2),), i32),
            pltpu.VMEM((6 * _PXT,), i32),
            pltpu.VMEM((_P,), i32),
            pltpu.VMEM((_P,), i32),
            pltpu.VMEM((_PXT,), i32),
            pltpu.VMEM((_PXT,), f32),
            pltpu.VMEM((_N_ITERS * _PXT,), i32),
            pltpu.VMEM((_N_ITERS * _PXT,), f32),
            pltpu.VMEM((_P, _C), f32),
            pltpu.VMEM((_P, _CP), f32),
            pltpu.VMEM((_P, _CP), f32),
            pltpu.VMEM((4 * _PXT,), i32),
            pltpu.VMEM((16 * _P,), f32),
            pltpu.VMEM_SHARED((_HW,), i32),
            pltpu.SemaphoreType.DMA,
            pltpu.SemaphoreType.DMA,
        ],
        compiler_params=pltpu.CompilerParams(needs_layout_passes=False),
    )
    out2 = pm(q2, k2, v2, pack0, dyx)
    return out2.reshape(B, H, W, C).transpose(0, 3, 1, 2)


# speedup vs baseline: 1.5779x; 1.0052x over previous
import functools

import jax
import jax.numpy as jnp
from jax import lax
from jax.experimental import pallas as pl
from jax.experimental.pallas import tpu as pltpu
from jax.experimental.pallas import tpu_sc as plsc

_N_ITERS = 5
_H = 224
_W = 224
_C = 192
_HW = _H * _W
_NSUB = 16
_ROWS_T = _H // _NSUB
_PXT = _ROWS_T * _W
_CP = 256
_P = 64
_NCH = _PXT // _P
_GPC = _P // 16
_NG = _PXT // 16
_NU = _C // 16


def _precompute_fields(B):
    radius0 = max(_H, _W) // 2
    pack0, dyx = [], []
    for i in range(B):
        pm_key = jax.random.fold_in(jax.random.key(42), i)
        ky, kx = jax.random.split(pm_key)
        sy = jax.random.randint(ky, (_H, _W), 0, _H)
        sx = jax.random.randint(kx, (_H, _W), 0, _W)
        pack0.append((sy * 256 + sx).astype(jnp.int32).ravel())
        per_it = []
        for it in range(_N_ITERS):
            r = max(radius0 >> it, 1)
            ss = []
            for s2 in range(2):
                kk = jax.random.fold_in(pm_key, it * 97 + s2 + 1)
                k1, k2 = jax.random.split(kk)
                dy = jax.random.randint(k1, (_H, _W), -r, r + 1)
                dx = jax.random.randint(k2, (_H, _W), -r, r + 1)
                ss.append(jnp.stack([dy.ravel(), dx.ravel()]))
            per_it.append(jnp.stack(ss))
        dyx.append(jnp.stack(per_it))
    pack0 = jnp.concatenate(pack0)
    dyx = jnp.stack(dyx)
    dyx = dyx.transpose(1, 2, 3, 0, 4).reshape(_N_ITERS * 2 * 2 * B * _HW)
    return pack0, dyx.astype(jnp.int32)


def _row_cost_acc(qb, kg, p):
    d0 = qb[p, pl.ds(0, 16)] - kg[p, pl.ds(0, 16)]
    acc = d0 * d0
    for u in range(1, _NU):
        d = qb[p, pl.ds(u * 16, 16)] - kg[p, pl.ds(u * 16, 16)]
        acc = acc + d * d
    return acc


def _pm_body(q_hbm, k_hbm, v_hbm, p0_hbm, dyx_hbm, out_hbm,
             st, cpack, fidx0, fidx1, cur, best, recp, recc, qb, kg0, kg1,
             dybuf, accT, state_sm, sem0, sem1):
    c = lax.axis_index("c")
    s = lax.axis_index("s")
    gbase = c * _HW + s * _PXT
    coff = c * _HW
    iota = lax.iota(jnp.int32, 16)

    fidxs = (fidx0, fidx1)
    kgs = (kg0, kg1)
    sems = (sem0, sem1)

    def unpack(p):
        return jnp.right_shift(p, 8), jnp.bitwise_and(p, 255)

    def start_gather(tbl, src, src_off, slot):
        fb = fidxs[slot]
        for g in range(_GPC):
            pk = src[pl.ds(src_off + g * 16, 16)]
            sy, sx = unpack(pk)
            fb[pl.ds(g * 16, 16)] = sy * _W + sx + coff
        return pltpu.async_copy(tbl.at[fb], kgs[slot], sems[slot])

    def partial_costs(kg):
        def pbody(p, _2):
            acc = _row_cost_acc(qb, kg, p)
            accT[pl.ds(p * 16, 16)] = acc
            return 0
        lax.fori_loop(0, _P, pbody, 0, unroll=4)

    def reduce_and_select(pbase, j, update):
        for g in range(_GPC):
            base16 = (g * 16 + iota) * 16
            tot = plsc.load_gather(accT, [base16])
            for lane in range(1, 16):
                tot = tot + plsc.load_gather(accT, [base16 + lane])
            lsl = pl.ds(pbase + g * 16, 16)
            if update:
                b0 = best[lsl]
                bet = tot < b0
                best[lsl] = jnp.where(bet, tot, b0)
                cur[lsl] = jnp.where(
                    bet, cpack[pl.ds(j * _PXT + pbase + g * 16, 16)],
                    cur[lsl])
            else:
                best[lsl] = tot

    def wait_slot(slot):
        pltpu.make_async_copy(k_hbm.at[fidxs[slot]], kgs[slot],
                              sems[slot]).wait()

    def cand_off(e):
        ch = e // 6
        j = e - ch * 6
        return j * _PXT + ch * _P, j, ch

    def eval_chunks_iter():
        off0, _j0, _c0 = cand_off(0)
        start_gather(k_hbm, cpack, off0, 0)

        def step(t, _):
            e0 = 2 * t
            for sub in range(2):
                e = e0 + sub
                off, j, ch = cand_off(e)
                pbase = ch * _P
                offn, _jn, _chn = cand_off(e + 1)
                start_gather(k_hbm, cpack,
                             jnp.minimum(offn, 6 * _PXT - _P), 1 - sub)
                @pl.when(j == 0)
                def _():
                    pltpu.sync_copy(q_hbm.at[pl.ds(gbase + pbase, _P)], qb)
                wait_slot(sub)
                partial_costs(kgs[sub])
                reduce_and_select(pbase, j, True)
            return 0
        lax.fori_loop(0, (_NCH * 6) // 2, step, 0)
        wait_slot(0)

    pltpu.sync_copy(p0_hbm.at[pl.ds(gbase, _PXT)], cur)
    start_gather(k_hbm, cur, 0, 0)

    def init_body(t, _):
        for sub in range(2):
            ch = 2 * t + sub
            pbase = ch * _P
            start_gather(k_hbm, cur,
                         jnp.minimum((ch + 1) * _P, _PXT - _P), 1 - sub)
            pltpu.sync_copy(q_hbm.at[pl.ds(gbase + pbase, _P)], qb)
            wait_slot(sub)
            partial_costs(kgs[sub])
            reduce_and_select(pbase, 0, False)
        return 0
    lax.fori_loop(0, _NCH // 2, init_body, 0)
    pbase_l = (_NCH - 1) * _P
    pltpu.sync_copy(q_hbm.at[pl.ds(gbase + pbase_l, _P)], qb)
    wait_slot(0)
    partial_costs(kg0)
    reduce_and_select(pbase_l, 0, False)
    pltpu.sync_copy(cur, state_sm.at[pl.ds(s * _PXT, _PXT)])
    plsc.subcore_barrier()

    def iteration(it, _):
        top = jnp.where(s == 0, _H - 1, s * _ROWS_T - 1)
        bot = jnp.where(s == _NSUB - 1, 0, (s + 1) * _ROWS_T)
        pltpu.sync_copy(state_sm.at[pl.ds(top * _W, _W)], st.at[pl.ds(0, _W)])
        pltpu.sync_copy(state_sm.at[pl.ds(s * _PXT, _PXT)],
                        st.at[pl.ds(_W, _PXT)])
        pltpu.sync_copy(state_sm.at[pl.ds(bot * _W, _W)],
                        st.at[pl.ds(_W + _PXT, _W)])
        plsc.subcore_barrier()

        for s2 in range(2):
            for comp in range(2):
                src = (it * 4 + s2 * 2 + comp) * (2 * _HW) + gbase
                pltpu.sync_copy(
                    dyx_hbm.at[pl.ds(src, _PXT)],
                    dybuf.at[pl.ds((s2 * 2 + comp) * _PXT, _PXT)])

        def cand_body(g, _2):
            lr = g // _ROWS_T
            gx = g - lr * _ROWS_T
            x0 = gx * 16
            introw = (lr + 1) * _W
            base = g * 16
            offm = iota + (x0 - 1)
            offm = jnp.where(offm < 0, offm + _W, offm)
            pL = plsc.load_gather(st, [introw + offm])
            syL, sxL = unpack(pL)
            cpack[pl.ds(0 * _PXT + base, 16)] = (
                jnp.left_shift(syL, 8) + jnp.minimum(sxL + 1, _W - 1))
            offp = iota + (x0 + 1)
            offp = jnp.where(offp > _W - 1, offp - _W, offp)
            pR = plsc.load_gather(st, [introw + offp])
            syR, sxR = unpack(pR)
            cpack[pl.ds(1 * _PXT + base, 16)] = (
                jnp.left_shift(syR, 8) + jnp.maximum(sxR - 1, 0))
            pU = st[pl.ds(lr * _W + x0, 16)]
            syU, sxU = unpack(pU)
            cpack[pl.ds(2 * _PXT + base, 16)] = (
                jnp.left_shift(jnp.minimum(syU + 1, _H - 1), 8) + sxU)
            pD = st[pl.ds((lr + 2) * _W + x0, 16)]
            syD, sxD = unpack(pD)
            cpack[pl.ds(3 * _PXT + base, 16)] = (
                jnp.left_shift(jnp.maximum(syD - 1, 0), 8) + sxD)
            pC = st[pl.ds(introw + x0, 16)]
            syC, sxC = unpack(pC)
            for s2 in range(2):
                dy = dybuf[pl.ds((s2 * 2 + 0) * _PXT + base, 16)]
                dx = dybuf[pl.ds((s2 * 2 + 1) * _PXT + base, 16)]
                cy = jnp.minimum(jnp.maximum(syC + dy, 0), _H - 1)
                cx = jnp.minimum(jnp.maximum(sxC + dx, 0), _W - 1)
                cpack[pl.ds((4 + s2) * _PXT + base, 16)] = (
                    jnp.left_shift(cy, 8) + cx)
            return 0
        lax.fori_loop(0, _NG, cand_body, 0)

        eval_chunks_iter()

        def rec_body(g, _2):
            recp[pl.ds(it * _PXT + g * 16, 16)] = cur[pl.ds(g * 16, 16)]
            recc[pl.ds(it * _PXT + g * 16, 16)] = best[pl.ds(g * 16, 16)]
            return 0
        lax.fori_loop(0, _NG, rec_body, 0)

        pltpu.sync_copy(cur, state_sm.at[pl.ds(s * _PXT, _PXT)])
        plsc.subcore_barrier()
        return 0
    lax.fori_loop(0, _N_ITERS, iteration, 0)

    def soft_body(g, _):
        base = g * 16
        cs = [recc[pl.ds(j * _PXT + base, 16)] for j in range(_N_ITERS)]
        m = cs[0]
        for j in range(1, _N_ITERS):
            m = jnp.minimum(m, cs[j])
        es = [jnp.exp(m - cj) for cj in cs]
        tot = es[0]
        for j in range(1, _N_ITERS):
            tot = tot + es[j]
        inv = 1.0 / tot
        for j in range(_N_ITERS):
            recc[pl.ds(j * _PXT + base, 16)] = es[j] * inv
        return 0
    lax.fori_loop(0, _NG, soft_body, 0)

    def obody(ch, _):
        pbase = ch * _P
        pend = start_gather(v_hbm, recp, 0 * _PXT + pbase, 0)
        for j in range(_N_ITERS):
            slot = j & 1
            nxt = (start_gather(v_hbm, recp, (j + 1) * _PXT + pbase,
                                1 - slot) if j < _N_ITERS - 1 else None)
            pend.wait()
            kg = kgs[slot]

            def px(p, _2, kg=kg, j=j):
                widx = jnp.broadcast_to(j * _PXT + pbase + p, (16,))
                w = plsc.load_gather(recc, [widx])
                for u in range(_NU):
                    sl = pl.ds(u * 16, 16)
                    if j == 0:
                        qb[p, sl] = w * kg[p, sl]
                    else:
                        qb[p, sl] = qb[p, sl] + w * kg[p, sl]
                return 0
            lax.fori_loop(0, _P, px, 0, unroll=4)
            pend = nxt
        pltpu.sync_copy(qb, out_hbm.at[pl.ds(gbase + pbase, _P)])
        return 0
    lax.fori_loop(0, _NCH, obody, 0)


def kernel(q, k, v):
    B, C, H, W = q.shape
    pack0, dyx = _precompute_fields(B)
    q2 = q.transpose(0, 2, 3, 1).reshape(B * _HW, C)
    k2 = jnp.pad(k.transpose(0, 2, 3, 1).reshape(B * _HW, C),
                 ((0, 0), (0, _CP - C)))
    v2 = jnp.pad(v.transpose(0, 2, 3, 1).reshape(B * _HW, C),
                 ((0, 0), (0, _CP - C)))

    mesh = plsc.VectorSubcoreMesh(core_axis_name="c", subcore_axis_name="s")
    f32, i32 = jnp.float32, jnp.int32
    pm = pl.kernel(
        _pm_body,
        mesh=mesh,
        out_type=jax.ShapeDtypeStruct((B * _HW, C), f32),
        scratch_types=[
            pltpu.VMEM((_W * (_ROWS_T + 2),), i32),
            pltpu.VMEM((6 * _PXT,), i32),
            pltpu.VMEM((_P,), i32),
            pltpu.VMEM((_P,), i32),
            pltpu.VMEM((_PXT,), i32),
            pltpu.VMEM((_PXT,), f32),
            pltpu.VMEM((_N_ITERS * _PXT,), i32),
            pltpu.VMEM((_N_ITERS * _PXT,), f32),
            pltpu.VMEM((_P, _C), f32),
            pltpu.VMEM((_P, _CP), f32),
            pltpu.VMEM((_P, _CP), f32),
            pltpu.VMEM((4 * _PXT,), i32),
            pltpu.VMEM((16 * _P,), f32),
            pltpu.VMEM_SHARED((_HW,), i32),
            pltpu.SemaphoreType.DMA,
            pltpu.SemaphoreType.DMA,
        ],
        compiler_params=pltpu.CompilerParams(needs_layout_passes=False),
    )
    out2 = pm(q2, k2, v2, pack0, dyx)
    return out2.reshape(B, H, W, C).transpose(0, 3, 1, 2)


# speedup vs baseline: 1.6003x; 1.0141x over previous
import functools

import jax
import jax.numpy as jnp
from jax import lax
from jax.experimental import pallas as pl
from jax.experimental.pallas import tpu as pltpu
from jax.experimental.pallas import tpu_sc as plsc

_N_ITERS = 5
_H = 224
_W = 224
_C = 192
_HW = _H * _W
_NSUB = 16
_ROWS_T = _H // _NSUB
_PXT = _ROWS_T * _W
_CP = 256
_P = 64
_NCH = _PXT // _P
_GPC = _P // 16
_NG = _PXT // 16
_NU = _C // 16


def _precompute_fields(B):
    radius0 = max(_H, _W) // 2
    pack0, dyx = [], []
    for i in range(B):
        pm_key = jax.random.fold_in(jax.random.key(42), i)
        ky, kx = jax.random.split(pm_key)
        sy = jax.random.randint(ky, (_H, _W), 0, _H)
        sx = jax.random.randint(kx, (_H, _W), 0, _W)
        pack0.append((sy * 256 + sx).astype(jnp.int32).ravel())
        per_it = []
        for it in range(_N_ITERS):
            r = max(radius0 >> it, 1)
            ss = []
            for s2 in range(2):
                kk = jax.random.fold_in(pm_key, it * 97 + s2 + 1)
                k1, k2 = jax.random.split(kk)
                dy = jax.random.randint(k1, (_H, _W), -r, r + 1)
                dx = jax.random.randint(k2, (_H, _W), -r, r + 1)
                ss.append(jnp.stack([dy.ravel(), dx.ravel()]))
            per_it.append(jnp.stack(ss))
        dyx.append(jnp.stack(per_it))
    pack0 = jnp.concatenate(pack0)
    dyx = jnp.stack(dyx)
    dyx = dyx.transpose(1, 2, 3, 0, 4).reshape(_N_ITERS * 2 * 2 * B * _HW)
    return pack0, dyx.astype(jnp.int32)


def _row_cost_acc(qb, kg, p):
    d0 = qb[p, pl.ds(0, 16)] - kg[p, pl.ds(0, 16)]
    acc = d0 * d0
    for u in range(1, _NU):
        d = qb[p, pl.ds(u * 16, 16)] - kg[p, pl.ds(u * 16, 16)]
        acc = acc + d * d
    return acc


def _pm_body(q_hbm, k_hbm, v_hbm, p0_hbm, dyx_hbm, out_hbm,
             st, cpack, fidx0, fidx1, cur, best, recp, recc, qb, kg0, kg1,
             dybuf, accT, state_sm, sem0, sem1):
    c = lax.axis_index("c")
    s = lax.axis_index("s")
    gbase = c * _HW + s * _PXT
    coff = c * _HW
    iota = lax.iota(jnp.int32, 16)

    fidxs = (fidx0, fidx1)
    kgs = (kg0, kg1)
    sems = (sem0, sem1)

    def unpack(p):
        return jnp.right_shift(p, 8), jnp.bitwise_and(p, 255)

    def start_gather(tbl, src, src_off, slot):
        fb = fidxs[slot]
        for g in range(_GPC):
            pk = src[pl.ds(src_off + g * 16, 16)]
            sy, sx = unpack(pk)
            fb[pl.ds(g * 16, 16)] = sy * _W + sx + coff
        return pltpu.async_copy(tbl.at[fb], kgs[slot], sems[slot])

    def partial_costs(kg):
        def pbody(p, _2):
            acc = _row_cost_acc(qb, kg, p)
            accT[pl.ds(p * 16, 16)] = acc
            return 0
        lax.fori_loop(0, _P, pbody, 0, unroll=4)

    def reduce_and_select(pbase, j, update):
        for g in range(_GPC):
            base16 = (g * 16 + iota) * 16
            tot = plsc.load_gather(accT, [base16])
            for lane in range(1, 16):
                tot = tot + plsc.load_gather(accT, [base16 + lane])
            lsl = pl.ds(pbase + g * 16, 16)
            if update:
                b0 = best[lsl]
                bet = tot < b0
                best[lsl] = jnp.where(bet, tot, b0)
                cur[lsl] = jnp.where(
                    bet, cpack[pl.ds(j * _PXT + pbase + g * 16, 16)],
                    cur[lsl])
            else:
                best[lsl] = tot

    def wait_slot(slot):
        pltpu.make_async_copy(k_hbm.at[fidxs[slot]], kgs[slot],
                              sems[slot]).wait()

    def cand_off(e):
        ch = e // 6
        j = e - ch * 6
        return j * _PXT + ch * _P, j, ch

    def eval_chunks_iter():
        off0, _j0, _c0 = cand_off(0)
        start_gather(k_hbm, cpack, off0, 0)

        def step(t, _):
            e0 = 2 * t
            for sub in range(2):
                e = e0 + sub
                off, j, ch = cand_off(e)
                pbase = ch * _P
                offn, _jn, _chn = cand_off(e + 1)
                start_gather(k_hbm, cpack,
                             jnp.minimum(offn, 6 * _PXT - _P), 1 - sub)
                @pl.when(j == 0)
                def _():
                    pltpu.sync_copy(q_hbm.at[pl.ds(gbase + pbase, _P)], qb)
                wait_slot(sub)
                partial_costs(kgs[sub])
                reduce_and_select(pbase, j, True)
            return 0
        lax.fori_loop(0, (_NCH * 6) // 2, step, 0)
        wait_slot(0)

    pltpu.sync_copy(p0_hbm.at[pl.ds(gbase, _PXT)], cur)
    start_gather(k_hbm, cur, 0, 0)

    def init_body(t, _):
        for sub in range(2):
            ch = 2 * t + sub
            pbase = ch * _P
            start_gather(k_hbm, cur,
                         jnp.minimum((ch + 1) * _P, _PXT - _P), 1 - sub)
            pltpu.sync_copy(q_hbm.at[pl.ds(gbase + pbase, _P)], qb)
            wait_slot(sub)
            partial_costs(kgs[sub])
            reduce_and_select(pbase, 0, False)
        return 0
    lax.fori_loop(0, _NCH // 2, init_body, 0)
    pbase_l = (_NCH - 1) * _P
    pltpu.sync_copy(q_hbm.at[pl.ds(gbase + pbase_l, _P)], qb)
    wait_slot(0)
    partial_costs(kg0)
    reduce_and_select(pbase_l, 0, False)
    pltpu.sync_copy(cur, state_sm.at[pl.ds(s * _PXT, _PXT)])
    plsc.subcore_barrier()

    def iteration(it, _):
        top = jnp.where(s == 0, _H - 1, s * _ROWS_T - 1)
        bot = jnp.where(s == _NSUB - 1, 0, (s + 1) * _ROWS_T)
        pltpu.sync_copy(state_sm.at[pl.ds(top * _W, _W)], st.at[pl.ds(0, _W)])
        pltpu.sync_copy(state_sm.at[pl.ds(s * _PXT, _PXT)],
                        st.at[pl.ds(_W, _PXT)])
        pltpu.sync_copy(state_sm.at[pl.ds(bot * _W, _W)],
                        st.at[pl.ds(_W + _PXT, _W)])
        plsc.subcore_barrier()

        for s2 in range(2):
            for comp in range(2):
                src = (it * 4 + s2 * 2 + comp) * (2 * _HW) + gbase
                pltpu.sync_copy(
                    dyx_hbm.at[pl.ds(src, _PXT)],
                    dybuf.at[pl.ds((s2 * 2 + comp) * _PXT, _PXT)])

        def cand_body(g, _2):
            lr = g // _ROWS_T
            gx = g - lr * _ROWS_T
            x0 = gx * 16
            introw = (lr + 1) * _W
            base = g * 16
            offm = iota + (x0 - 1)
            offm = jnp.where(offm < 0, offm + _W, offm)
            pL = plsc.load_gather(st, [introw + offm])
            syL, sxL = unpack(pL)
            cpack[pl.ds(0 * _PXT + base, 16)] = (
                jnp.left_shift(syL, 8) + jnp.minimum(sxL + 1, _W - 1))
            offp = iota + (x0 + 1)
            offp = jnp.where(offp > _W - 1, offp - _W, offp)
            pR = plsc.load_gather(st, [introw + offp])
            syR, sxR = unpack(pR)
            cpack[pl.ds(1 * _PXT + base, 16)] = (
                jnp.left_shift(syR, 8) + jnp.maximum(sxR - 1, 0))
            pU = st[pl.ds(lr * _W + x0, 16)]
            syU, sxU = unpack(pU)
            cpack[pl.ds(2 * _PXT + base, 16)] = (
                jnp.left_shift(jnp.minimum(syU + 1, _H - 1), 8) + sxU)
            pD = st[pl.ds((lr + 2) * _W + x0, 16)]
            syD, sxD = unpack(pD)
            cpack[pl.ds(3 * _PXT + base, 16)] = (
                jnp.left_shift(jnp.maximum(syD - 1, 0), 8) + sxD)
            pC = st[pl.ds(introw + x0, 16)]
            syC, sxC = unpack(pC)
            for s2 in range(2):
                dy = dybuf[pl.ds((s2 * 2 + 0) * _PXT + base, 16)]
                dx = dybuf[pl.ds((s2 * 2 + 1) * _PXT + base, 16)]
                cy = jnp.minimum(jnp.maximum(syC + dy, 0), _H - 1)
                cx = jnp.minimum(jnp.maximum(sxC + dx, 0), _W - 1)
                cpack[pl.ds((4 + s2) * _PXT + base, 16)] = (
                    jnp.left_shift(cy, 8) + cx)
            return 0
        lax.fori_loop(0, _NG, cand_body, 0)

        eval_chunks_iter()

        def rec_body(g, _2):
            recp[pl.ds(it * _PXT + g * 16, 16)] = cur[pl.ds(g * 16, 16)]
            recc[pl.ds(it * _PXT + g * 16, 16)] = best[pl.ds(g * 16, 16)]
            return 0
        lax.fori_loop(0, _NG, rec_body, 0)

        pltpu.sync_copy(cur, state_sm.at[pl.ds(s * _PXT, _PXT)])
        plsc.subcore_barrier()
        return 0
    lax.fori_loop(0, _N_ITERS, iteration, 0)

    def soft_body(g, _):
        base = g * 16
        cs = [recc[pl.ds(j * _PXT + base, 16)] for j in range(_N_ITERS)]
        m = cs[0]
        for j in range(1, _N_ITERS):
            m = jnp.minimum(m, cs[j])
        es = [jnp.exp(m - cj) for cj in cs]
        tot = es[0]
        for j in range(1, _N_ITERS):
            tot = tot + es[j]
        inv = 1.0 / tot
        for j in range(_N_ITERS):
            recc[pl.ds(j * _PXT + base, 16)] = es[j] * inv
        return 0
    lax.fori_loop(0, _NG, soft_body, 0)

    def comb_compute(pbase, j, slot):
        kg = kgs[slot]

        def px(p, _2, kg=kg, j=j):
            widx = jnp.broadcast_to(j * _PXT + pbase + p, (16,))
            w = plsc.load_gather(recc, [widx])
            for u in range(_NU):
                sl = pl.ds(u * 16, 16)
                if j == 0:
                    qb[p, sl] = w * kg[p, sl]
                else:
                    qb[p, sl] = qb[p, sl] + w * kg[p, sl]
            return 0
        lax.fori_loop(0, _P, px, 0, unroll=4)

    start_gather(v_hbm, recp, 0, 0)

    def pair_body(t, _):
        ch0 = 2 * t
        for i in range(10):
            j = i % 5
            ch = ch0 + (i // 5)
            pbase = ch * _P
            jn = (i + 1) % 5
            chn = ch0 + ((i + 1) // 5)
            offn = jn * _PXT + jnp.minimum(chn, _NCH - 1) * _P
            start_gather(v_hbm, recp, offn, 1 - (i & 1))
            wait_slot(i & 1)
            comb_compute(pbase, j, i & 1)
            if j == _N_ITERS - 1:
                pltpu.sync_copy(qb, out_hbm.at[pl.ds(gbase + pbase, _P)])
        return 0
    lax.fori_loop(0, (_NCH - 1) // 2, pair_body, 0)
    pbase_t = (_NCH - 1) * _P
    for i in range(_N_ITERS):
        offn = min(i + 1, _N_ITERS - 1) * _PXT + pbase_t
        start_gather(v_hbm, recp, offn, 1 - (i & 1))
        wait_slot(i & 1)
        comb_compute(pbase_t, i, i & 1)
    pltpu.sync_copy(qb, out_hbm.at[pl.ds(gbase + pbase_t, _P)])
    wait_slot(1)


def kernel(q, k, v):
    B, C, H, W = q.shape
    pack0, dyx = _precompute_fields(B)
    q2 = q.transpose(0, 2, 3, 1).reshape(B * _HW, C)
    k2 = jnp.pad(k.transpose(0, 2, 3, 1).reshape(B * _HW, C),
                 ((0, 0), (0, _CP - C)))
    v2 = jnp.pad(v.transpose(0, 2, 3, 1).reshape(B * _HW, C),
                 ((0, 0), (0, _CP - C)))

    mesh = plsc.VectorSubcoreMesh(core_axis_name="c", subcore_axis_name="s")
    f32, i32 = jnp.float32, jnp.int32
    pm = pl.kernel(
        _pm_body,
        mesh=mesh,
        out_type=jax.ShapeDtypeStruct((B * _HW, C), f32),
        scratch_types=[
            pltpu.VMEM((_W * (_ROWS_T + 2),), i32),
            pltpu.VMEM((6 * _PXT,), i32),
            pltpu.VMEM((_P,), i32),
            pltpu.VMEM((_P,), i32),
            pltpu.VMEM((_PXT,), i32),
            pltpu.VMEM((_PXT,), f32),
            pltpu.VMEM((_N_ITERS * _PXT,), i32),
            pltpu.VMEM((_N_ITERS * _PXT,), f32),
            pltpu.VMEM((_P, _C), f32),
            pltpu.VMEM((_P, _CP), f32),
            pltpu.VMEM((_P, _CP), f32),
            pltpu.VMEM((4 * _PXT,), i32),
            pltpu.VMEM((16 * _P,), f32),
            pltpu.VMEM_SHARED((_HW,), i32),
            pltpu.SemaphoreType.DMA,
            pltpu.SemaphoreType.DMA,
        ],
        compiler_params=pltpu.CompilerParams(needs_layout_passes=False),
    )
    out2 = pm(q2, k2, v2, pack0, dyx)
    return out2.reshape(B, H, W, C).transpose(0, 3, 1, 2)
